# Initial kernel scaffold; baseline (speedup 1.0000x reference)
import jax
import jax.numpy as jnp
from jax.experimental import pallas as pl


def kernel(hidden_states, gate_proj_w, gate_weights, up_weights, down_weights):
    raise NotImplementedError("write your pallas kernel here")


# speedup vs baseline: 1.3845x; 1.3845x over previous
import functools

import jax
import jax.numpy as jnp
from jax import lax
from jax.experimental import pallas as pl
from jax.experimental.pallas import tpu as pltpu
from jax.experimental.pallas import tpu_sc as plsc

TOKENS = 2048
HID = 1024
INTER = 1024
NEXP = 8
BLK = 128
CAP = 2 * TOKENS + NEXP * BLK
NT = CAP // BLK
RT = 256
NRT = TOKENS // RT

NC = 2
NS = 16
NW = NC * NS
CHUNK = TOKENS // NW
SUB = CHUNK // 2
WPAD = 128

_F32 = jnp.float32
_I32 = jnp.int32


def _router_body(x_ref, gw_ref, e0_ref, e1_ref, w0_ref, w1_ref, r0_ref,
                 r1_ref, cnt_ref, carry):
    i = pl.program_id(0)

    @pl.when(i == 0)
    def _():
        carry[...] = jnp.zeros_like(carry)

    x = x_ref[...]
    gw = gw_ref[...]
    logits = lax.dot_general(x, gw, (((1,), (1,)), ((), ())),
                             preferred_element_type=_F32)
    m = jnp.max(logits, axis=1, keepdims=True)
    ex = jnp.exp(logits - m)
    probs = ex / jnp.sum(ex, axis=1, keepdims=True)

    lane = lax.broadcasted_iota(_I32, (RT, NEXP), 1)
    p0 = jnp.max(probs, axis=1, keepdims=True)
    a0 = jnp.min(jnp.where(probs == p0, lane, NEXP), axis=1, keepdims=True)
    oh0 = lane == a0
    probs1 = jnp.where(oh0, -jnp.inf, probs)
    p1 = jnp.max(probs1, axis=1, keepdims=True)
    a1 = jnp.min(jnp.where(probs1 == p1, lane, NEXP), axis=1, keepdims=True)
    oh1 = lane == a1
    s = p0 + p1
    w0 = p0 / s
    w1 = p1 / s

    mm = (oh0 | oh1).astype(_F32)
    row = lax.broadcasted_iota(_I32, (RT, RT), 0)
    col = lax.broadcasted_iota(_I32, (RT, RT), 1)
    ltri = (col < row).astype(_F32)
    csum = lax.dot_general(ltri, mm, (((1,), (0,)), ((), ())),
                           preferred_element_type=_F32)
    rank = csum + carry[...]

    e0_ref[...] = a0
    e1_ref[...] = a1
    w0_ref[...] = jnp.broadcast_to(w0, (RT, WPAD))
    w1_ref[...] = jnp.broadcast_to(w1, (RT, WPAD))
    r0_ref[...] = jnp.sum(jnp.where(oh0, rank, 0.0), axis=1, keepdims=True)
    r1_ref[...] = jnp.sum(jnp.where(oh1, rank, 0.0), axis=1, keepdims=True)

    new_carry = carry[...] + jnp.sum(mm, axis=0, keepdims=True)
    carry[...] = new_carry
    cnt_ref[...] = new_carry.astype(_I32)


_router = pl.pallas_call(
    _router_body,
    grid=(NRT,),
    in_specs=[
        pl.BlockSpec((RT, HID), lambda i: (i, 0)),
        pl.BlockSpec((NEXP, HID), lambda i: (0, 0)),
    ],
    out_specs=[
        pl.BlockSpec((RT, 1), lambda i: (i, 0)),
        pl.BlockSpec((RT, 1), lambda i: (i, 0)),
        pl.BlockSpec((RT, WPAD), lambda i: (i, 0)),
        pl.BlockSpec((RT, WPAD), lambda i: (i, 0)),
        pl.BlockSpec((RT, 1), lambda i: (i, 0)),
        pl.BlockSpec((RT, 1), lambda i: (i, 0)),
        pl.BlockSpec((1, NEXP), lambda i: (0, 0)),
    ],
    out_shape=[
        jax.ShapeDtypeStruct((TOKENS, 1), _I32),
        jax.ShapeDtypeStruct((TOKENS, 1), _I32),
        jax.ShapeDtypeStruct((TOKENS, WPAD), _F32),
        jax.ShapeDtypeStruct((TOKENS, WPAD), _F32),
        jax.ShapeDtypeStruct((TOKENS, 1), _F32),
        jax.ShapeDtypeStruct((TOKENS, 1), _F32),
        jax.ShapeDtypeStruct((1, NEXP), _I32),
    ],
    scratch_shapes=[pltpu.VMEM((1, NEXP), _F32)],
)


def _finalize_body(cnt_ref, e0_ref, e1_ref, r0_ref, r1_ref, d0_ref, d1_ref,
                   map_ref):
    cnt = cnt_ref[...]
    padded = ((cnt + (BLK - 1)) >> 7) << 7
    pf = padded.astype(_F32)
    r8 = lax.broadcasted_iota(_I32, (NEXP, NEXP), 0)
    c8 = lax.broadcasted_iota(_I32, (NEXP, NEXP), 1)
    ut = (r8 < c8).astype(_F32)
    pstart = lax.dot_general(pf, ut, (((1,), (0,)), ((), ())),
                             preferred_element_type=_F32)

    lane = lax.broadcasted_iota(_I32, (RT, NEXP), 1)
    e0 = e0_ref[...]
    e1 = e1_ref[...]
    ps0 = jnp.sum(jnp.where(lane == e0, pstart, 0.0), axis=1, keepdims=True)
    ps1 = jnp.sum(jnp.where(lane == e1, pstart, 0.0), axis=1, keepdims=True)
    d0_ref[...] = (r0_ref[...] + ps0).astype(_I32)
    d1_ref[...] = (r1_ref[...] + ps1).astype(_I32)

    tb = (lax.broadcasted_iota(_I32, (NT, 1), 0) * BLK).astype(_F32)
    psb = jnp.broadcast_to(pstart, (NT, NEXP))
    acc = jnp.sum((psb <= tb).astype(_I32), axis=1, keepdims=True)
    map_ref[...] = jnp.clip(acc - 1, 0, NEXP - 1)


_finalize = pl.pallas_call(
    _finalize_body,
    grid=(NRT,),
    in_specs=[
        pl.BlockSpec((1, NEXP), lambda i: (0, 0)),
        pl.BlockSpec((RT, 1), lambda i: (i, 0)),
        pl.BlockSpec((RT, 1), lambda i: (i, 0)),
        pl.BlockSpec((RT, 1), lambda i: (i, 0)),
        pl.BlockSpec((RT, 1), lambda i: (i, 0)),
    ],
    out_specs=[
        pl.BlockSpec((RT, 1), lambda i: (i, 0)),
        pl.BlockSpec((RT, 1), lambda i: (i, 0)),
        pl.BlockSpec((NT, 1), lambda i: (0, 0)),
    ],
    out_shape=[
        jax.ShapeDtypeStruct((TOKENS, 1), _I32),
        jax.ShapeDtypeStruct((TOKENS, 1), _I32),
        jax.ShapeDtypeStruct((NT, 1), _I32),
    ],
)


@functools.lru_cache(maxsize=None)
def _sc_kernels():
    mesh = plsc.VectorSubcoreMesh(core_axis_name="c", subcore_axis_name="s",
                                  num_cores=NC, num_subcores=NS)

    @functools.partial(
        pl.kernel,
        out_type=(
            jax.ShapeDtypeStruct((CAP, HID), _F32),
            jax.ShapeDtypeStruct((CAP, WPAD), _F32),
        ),
        mesh=mesh,
        scratch_types=[
            pltpu.VMEM((CHUNK,), _I32),
            pltpu.VMEM((CHUNK,), _I32),
            pltpu.VMEM((CHUNK, HID), _F32),
            pltpu.VMEM((CHUNK, WPAD), _F32),
            pltpu.VMEM((CHUNK, WPAD), _F32),
        ],
    )
    def _dispatch(x_hbm, d0_hbm, d1_hbm, w0_hbm, w1_hbm, xs_hbm, ws_hbm,
                  idx0_v, idx1_v, rows_v, w0_v, w1_v):
        wid = lax.axis_index("s") * NC + lax.axis_index("c")
        base = wid * CHUNK
        pltpu.sync_copy(d0_hbm.at[pl.ds(base, CHUNK)], idx0_v)
        pltpu.sync_copy(d1_hbm.at[pl.ds(base, CHUNK)], idx1_v)
        pltpu.sync_copy(x_hbm.at[pl.ds(base, CHUNK), :], rows_v)
        pltpu.sync_copy(w0_hbm.at[pl.ds(base, CHUNK), :], w0_v)
        pltpu.sync_copy(w1_hbm.at[pl.ds(base, CHUNK), :], w1_v)
        pltpu.sync_copy(rows_v, xs_hbm.at[idx0_v])
        pltpu.sync_copy(rows_v, xs_hbm.at[idx1_v])
        pltpu.sync_copy(w0_v, ws_hbm.at[idx0_v])
        pltpu.sync_copy(w1_v, ws_hbm.at[idx1_v])

    @functools.partial(
        pl.kernel,
        out_type=jax.ShapeDtypeStruct((TOKENS, HID), _F32),
        mesh=mesh,
        scratch_types=[
            pltpu.VMEM((SUB,), _I32),
            pltpu.VMEM((SUB,), _I32),
            pltpu.VMEM((SUB, HID), _F32),
            pltpu.VMEM((SUB, HID), _F32),
        ],
    )
    def _combine(y_hbm, d0_hbm, d1_hbm, out_hbm,
                 idx0_v, idx1_v, rows0_v, rows1_v):
        wid = lax.axis_index("s") * NC + lax.axis_index("c")
        for h in range(CHUNK // SUB):
            base = wid * CHUNK + h * SUB
            pltpu.sync_copy(d0_hbm.at[pl.ds(base, SUB)], idx0_v)
            pltpu.sync_copy(d1_hbm.at[pl.ds(base, SUB)], idx1_v)
            pltpu.sync_copy(y_hbm.at[idx0_v], rows0_v)
            pltpu.sync_copy(y_hbm.at[idx1_v], rows1_v)
            for t in range(SUB):
                def _add16(j, _, t=t):
                    sl = pl.ds(j * 16, 16)
                    rows0_v[t, sl] = rows0_v[t, sl] + rows1_v[t, sl]
                    return 0
                lax.fori_loop(0, HID // 16, _add16, 0)
            pltpu.sync_copy(rows0_v, out_hbm.at[pl.ds(base, SUB), :])

    return _dispatch, _combine


def _ffn_body(m_ref, xs_ref, wg_ref, wu_ref, wd_ref, ws_ref, y_ref):
    del m_ref
    x = xs_ref[...]
    g = jnp.dot(x, wg_ref[0], preferred_element_type=_F32)
    u = jnp.dot(x, wu_ref[0], preferred_element_type=_F32)
    inter = g * jax.nn.sigmoid(g) * u
    y = jnp.dot(inter, wd_ref[0], preferred_element_type=_F32)
    w = ws_ref[...][:, :1]
    y_ref[...] = y * w


_ffn = pl.pallas_call(
    _ffn_body,
    grid_spec=pltpu.PrefetchScalarGridSpec(
        num_scalar_prefetch=1,
        grid=(NT,),
        in_specs=[
            pl.BlockSpec((BLK, HID), lambda i, m: (i, 0)),
            pl.BlockSpec((1, HID, INTER), lambda i, m: (m[i], 0, 0)),
            pl.BlockSpec((1, HID, INTER), lambda i, m: (m[i], 0, 0)),
            pl.BlockSpec((1, INTER, HID), lambda i, m: (m[i], 0, 0)),
            pl.BlockSpec((BLK, WPAD), lambda i, m: (i, 0)),
        ],
        out_specs=pl.BlockSpec((BLK, HID), lambda i, m: (i, 0)),
    ),
    out_shape=jax.ShapeDtypeStruct((CAP, HID), _F32),
)


def kernel(hidden_states, gate_proj_w, gate_weights, up_weights, down_weights):
    dispatch, combine = _sc_kernels()
    e0, e1, w0b, w1b, r0, r1, cnt = _router(hidden_states, gate_proj_w)
    d0, d1, emap = _finalize(cnt, e0, e1, r0, r1)
    d0f = d0.reshape(TOKENS)
    d1f = d1.reshape(TOKENS)
    xs, ws = dispatch(hidden_states, d0f, d1f, w0b, w1b)
    y = _ffn(emap.reshape(NT), xs, gate_weights, up_weights, down_weights, ws)
    return combine(y, d0f, d1f)


# speedup vs baseline: 1.4031x; 1.0134x over previous
import functools

import jax
import jax.numpy as jnp
from jax import lax
from jax.experimental import pallas as pl
from jax.experimental.pallas import tpu as pltpu
from jax.experimental.pallas import tpu_sc as plsc

TOKENS = 2048
HID = 1024
INTER = 1024
NEXP = 8
BLK = 256
BSH = 8
CAP = 2 * TOKENS + NEXP * BLK
NT = CAP // BLK
RT = 256
NRT = TOKENS // RT

NC = 2
NS = 16
NW = NC * NS
CHUNK = TOKENS // NW
SUB = CHUNK // 2
WPAD = 128

_F32 = jnp.float32
_I32 = jnp.int32


def _router_body(x_ref, gw_ref, e0_ref, e1_ref, w0_ref, w1_ref, r0_ref,
                 r1_ref, cnt_ref, carry):
    i = pl.program_id(0)

    @pl.when(i == 0)
    def _():
        carry[...] = jnp.zeros_like(carry)

    x = x_ref[...]
    gw = gw_ref[...]
    logits = lax.dot_general(x, gw, (((1,), (1,)), ((), ())),
                             preferred_element_type=_F32)
    m = jnp.max(logits, axis=1, keepdims=True)
    ex = jnp.exp(logits - m)
    probs = ex / jnp.sum(ex, axis=1, keepdims=True)

    lane = lax.broadcasted_iota(_I32, (RT, NEXP), 1)
    p0 = jnp.max(probs, axis=1, keepdims=True)
    a0 = jnp.min(jnp.where(probs == p0, lane, NEXP), axis=1, keepdims=True)
    oh0 = lane == a0
    probs1 = jnp.where(oh0, -jnp.inf, probs)
    p1 = jnp.max(probs1, axis=1, keepdims=True)
    a1 = jnp.min(jnp.where(probs1 == p1, lane, NEXP), axis=1, keepdims=True)
    oh1 = lane == a1
    s = p0 + p1
    w0 = p0 / s
    w1 = p1 / s

    mm = (oh0 | oh1).astype(_F32)
    row = lax.broadcasted_iota(_I32, (RT, RT), 0)
    col = lax.broadcasted_iota(_I32, (RT, RT), 1)
    ltri = (col < row).astype(_F32)
    csum = lax.dot_general(ltri, mm, (((1,), (0,)), ((), ())),
                           preferred_element_type=_F32)
    rank = csum + carry[...]

    e0_ref[...] = a0
    e1_ref[...] = a1
    w0_ref[...] = jnp.broadcast_to(w0, (RT, WPAD))
    w1_ref[...] = jnp.broadcast_to(w1, (RT, WPAD))
    r0_ref[...] = jnp.sum(jnp.where(oh0, rank, 0.0), axis=1, keepdims=True)
    r1_ref[...] = jnp.sum(jnp.where(oh1, rank, 0.0), axis=1, keepdims=True)

    new_carry = carry[...] + jnp.sum(mm, axis=0, keepdims=True)
    carry[...] = new_carry
    cnt_ref[...] = new_carry.astype(_I32)


_router = pl.pallas_call(
    _router_body,
    grid=(NRT,),
    in_specs=[
        pl.BlockSpec((RT, HID), lambda i: (i, 0)),
        pl.BlockSpec((NEXP, HID), lambda i: (0, 0)),
    ],
    out_specs=[
        pl.BlockSpec((RT, 1), lambda i: (i, 0)),
        pl.BlockSpec((RT, 1), lambda i: (i, 0)),
        pl.BlockSpec((RT, WPAD), lambda i: (i, 0)),
        pl.BlockSpec((RT, WPAD), lambda i: (i, 0)),
        pl.BlockSpec((RT, 1), lambda i: (i, 0)),
        pl.BlockSpec((RT, 1), lambda i: (i, 0)),
        pl.BlockSpec((1, NEXP), lambda i: (0, 0)),
    ],
    out_shape=[
        jax.ShapeDtypeStruct((TOKENS, 1), _I32),
        jax.ShapeDtypeStruct((TOKENS, 1), _I32),
        jax.ShapeDtypeStruct((TOKENS, WPAD), _F32),
        jax.ShapeDtypeStruct((TOKENS, WPAD), _F32),
        jax.ShapeDtypeStruct((TOKENS, 1), _F32),
        jax.ShapeDtypeStruct((TOKENS, 1), _F32),
        jax.ShapeDtypeStruct((1, NEXP), _I32),
    ],
    scratch_shapes=[pltpu.VMEM((1, NEXP), _F32)],
)


def _finalize_body(cnt_ref, e0_ref, e1_ref, r0_ref, r1_ref, d0_ref, d1_ref,
                   map_ref):
    cnt = cnt_ref[...]
    padded = ((cnt + (BLK - 1)) >> BSH) << BSH
    pf = padded.astype(_F32)
    r8 = lax.broadcasted_iota(_I32, (NEXP, NEXP), 0)
    c8 = lax.broadcasted_iota(_I32, (NEXP, NEXP), 1)
    ut = (r8 < c8).astype(_F32)
    pstart = lax.dot_general(pf, ut, (((1,), (0,)), ((), ())),
                             preferred_element_type=_F32)

    lane = lax.broadcasted_iota(_I32, (RT, NEXP), 1)
    e0 = e0_ref[...]
    e1 = e1_ref[...]
    ps0 = jnp.sum(jnp.where(lane == e0, pstart, 0.0), axis=1, keepdims=True)
    ps1 = jnp.sum(jnp.where(lane == e1, pstart, 0.0), axis=1, keepdims=True)
    d0_ref[...] = (r0_ref[...] + ps0).astype(_I32)
    d1_ref[...] = (r1_ref[...] + ps1).astype(_I32)

    tb = (lax.broadcasted_iota(_I32, (NT, 1), 0) * BLK).astype(_F32)
    psb = jnp.broadcast_to(pstart, (NT, NEXP))
    acc = jnp.sum((psb <= tb).astype(_I32), axis=1, keepdims=True)
    map_ref[...] = jnp.clip(acc - 1, 0, NEXP - 1)


_finalize = pl.pallas_call(
    _finalize_body,
    grid=(NRT,),
    in_specs=[
        pl.BlockSpec((1, NEXP), lambda i: (0, 0)),
        pl.BlockSpec((RT, 1), lambda i: (i, 0)),
        pl.BlockSpec((RT, 1), lambda i: (i, 0)),
        pl.BlockSpec((RT, 1), lambda i: (i, 0)),
        pl.BlockSpec((RT, 1), lambda i: (i, 0)),
    ],
    out_specs=[
        pl.BlockSpec((RT, 1), lambda i: (i, 0)),
        pl.BlockSpec((RT, 1), lambda i: (i, 0)),
        pl.BlockSpec((NT, 1), lambda i: (0, 0)),
    ],
    out_shape=[
        jax.ShapeDtypeStruct((TOKENS, 1), _I32),
        jax.ShapeDtypeStruct((TOKENS, 1), _I32),
        jax.ShapeDtypeStruct((NT, 1), _I32),
    ],
)


@functools.lru_cache(maxsize=None)
def _sc_kernels():
    mesh = plsc.VectorSubcoreMesh(core_axis_name="c", subcore_axis_name="s",
                                  num_cores=NC, num_subcores=NS)

    @functools.partial(
        pl.kernel,
        out_type=(
            jax.ShapeDtypeStruct((CAP, HID), _F32),
            jax.ShapeDtypeStruct((CAP, WPAD), _F32),
        ),
        mesh=mesh,
        scratch_types=[
            pltpu.VMEM((CHUNK,), _I32),
            pltpu.VMEM((CHUNK,), _I32),
            pltpu.VMEM((CHUNK, HID), _F32),
            pltpu.VMEM((CHUNK, WPAD), _F32),
            pltpu.VMEM((CHUNK, WPAD), _F32),
        ],
    )
    def _dispatch(x_hbm, d0_hbm, d1_hbm, w0_hbm, w1_hbm, xs_hbm, ws_hbm,
                  idx0_v, idx1_v, rows_v, w0_v, w1_v):
        wid = lax.axis_index("s") * NC + lax.axis_index("c")
        base = wid * CHUNK
        pltpu.sync_copy(d0_hbm.at[pl.ds(base, CHUNK)], idx0_v)
        pltpu.sync_copy(d1_hbm.at[pl.ds(base, CHUNK)], idx1_v)
        pltpu.sync_copy(x_hbm.at[pl.ds(base, CHUNK), :], rows_v)
        pltpu.sync_copy(w0_hbm.at[pl.ds(base, CHUNK), :], w0_v)
        pltpu.sync_copy(w1_hbm.at[pl.ds(base, CHUNK), :], w1_v)
        pltpu.sync_copy(rows_v, xs_hbm.at[idx0_v])
        pltpu.sync_copy(rows_v, xs_hbm.at[idx1_v])
        pltpu.sync_copy(w0_v, ws_hbm.at[idx0_v])
        pltpu.sync_copy(w1_v, ws_hbm.at[idx1_v])

    @functools.partial(
        pl.kernel,
        out_type=jax.ShapeDtypeStruct((TOKENS, HID), _F32),
        mesh=mesh,
        scratch_types=[
            pltpu.VMEM((SUB,), _I32),
            pltpu.VMEM((SUB,), _I32),
            pltpu.VMEM((SUB, HID), _F32),
            pltpu.VMEM((SUB, HID), _F32),
        ],
    )
    def _combine(y_hbm, d0_hbm, d1_hbm, out_hbm,
                 idx0_v, idx1_v, rows0_v, rows1_v):
        wid = lax.axis_index("s") * NC + lax.axis_index("c")
        for h in range(CHUNK // SUB):
            base = wid * CHUNK + h * SUB
            pltpu.sync_copy(d0_hbm.at[pl.ds(base, SUB)], idx0_v)
            pltpu.sync_copy(d1_hbm.at[pl.ds(base, SUB)], idx1_v)
            pltpu.sync_copy(y_hbm.at[idx0_v], rows0_v)
            pltpu.sync_copy(y_hbm.at[idx1_v], rows1_v)
            for t in range(SUB):
                def _add16(j, _, t=t):
                    sl = pl.ds(j * 16, 16)
                    rows0_v[t, sl] = rows0_v[t, sl] + rows1_v[t, sl]
                    return 0
                lax.fori_loop(0, HID // 16, _add16, 0)
            pltpu.sync_copy(rows0_v, out_hbm.at[pl.ds(base, SUB), :])

    return _dispatch, _combine


def _ffn_body(m_ref, xs_ref, wg_ref, wu_ref, wd_ref, ws_ref, y_ref):
    del m_ref
    x = xs_ref[...]
    g = jnp.dot(x, wg_ref[0], preferred_element_type=_F32)
    u = jnp.dot(x, wu_ref[0], preferred_element_type=_F32)
    inter = g * jax.nn.sigmoid(g) * u
    y = jnp.dot(inter, wd_ref[0], preferred_element_type=_F32)
    w = ws_ref[...][:, :1]
    y_ref[...] = y * w


_ffn = pl.pallas_call(
    _ffn_body,
    grid_spec=pltpu.PrefetchScalarGridSpec(
        num_scalar_prefetch=1,
        grid=(NT,),
        in_specs=[
            pl.BlockSpec((BLK, HID), lambda i, m: (i, 0)),
            pl.BlockSpec((1, HID, INTER), lambda i, m: (m[i], 0, 0)),
            pl.BlockSpec((1, HID, INTER), lambda i, m: (m[i], 0, 0)),
            pl.BlockSpec((1, INTER, HID), lambda i, m: (m[i], 0, 0)),
            pl.BlockSpec((BLK, WPAD), lambda i, m: (i, 0)),
        ],
        out_specs=pl.BlockSpec((BLK, HID), lambda i, m: (i, 0)),
    ),
    out_shape=jax.ShapeDtypeStruct((CAP, HID), _F32),
)


def kernel(hidden_states, gate_proj_w, gate_weights, up_weights, down_weights):
    dispatch, combine = _sc_kernels()
    e0, e1, w0b, w1b, r0, r1, cnt = _router(hidden_states, gate_proj_w)
    d0, d1, emap = _finalize(cnt, e0, e1, r0, r1)
    d0f = d0.reshape(TOKENS)
    d1f = d1.reshape(TOKENS)
    xs, ws = dispatch(hidden_states, d0f, d1f, w0b, w1b)
    y = _ffn(emap.reshape(NT), xs, gate_weights, up_weights, down_weights, ws)
    return combine(y, d0f, d1f)


# speedup vs baseline: 1.5199x; 1.0833x over previous
import functools

import jax
import jax.numpy as jnp
from jax import lax
from jax.experimental import pallas as pl
from jax.experimental.pallas import tpu as pltpu
from jax.experimental.pallas import tpu_sc as plsc

TOKENS = 2048
HID = 1024
INTER = 1024
NEXP = 8
BLK = 256
BSH = 8
CAP = 2 * TOKENS + NEXP * BLK
NT = CAP // BLK
RT = 256
NRT = TOKENS // RT
WPAD = 128

NC = 2
NS = 16
NW = NC * NS
CHUNK = TOKENS // NW
SUB = CHUNK // 2
LANES = 16

_F32 = jnp.float32
_I32 = jnp.int32


def _router_body(x_ref, gw_ref, e0_ref, e1_ref, w0_ref, w1_ref, r0_ref,
                 r1_ref, ps_ref, map_ref, carry):
    i = pl.program_id(0)

    @pl.when(i == 0)
    def _():
        carry[...] = jnp.zeros_like(carry)

    x = x_ref[...]
    gw = gw_ref[...]
    logits = lax.dot_general(x, gw, (((1,), (1,)), ((), ())),
                             preferred_element_type=_F32)
    m = jnp.max(logits, axis=1, keepdims=True)
    ex = jnp.exp(logits - m)
    probs = ex / jnp.sum(ex, axis=1, keepdims=True)

    lane = lax.broadcasted_iota(_I32, (RT, NEXP), 1)
    p0 = jnp.max(probs, axis=1, keepdims=True)
    a0 = jnp.min(jnp.where(probs == p0, lane, NEXP), axis=1, keepdims=True)
    oh0 = lane == a0
    probs1 = jnp.where(oh0, -jnp.inf, probs)
    p1 = jnp.max(probs1, axis=1, keepdims=True)
    a1 = jnp.min(jnp.where(probs1 == p1, lane, NEXP), axis=1, keepdims=True)
    oh1 = lane == a1
    s = p0 + p1
    w0 = p0 / s
    w1 = p1 / s

    mm = (oh0 | oh1).astype(_F32)
    row = lax.broadcasted_iota(_I32, (RT, RT), 0)
    col = lax.broadcasted_iota(_I32, (RT, RT), 1)
    ltri = (col < row).astype(_F32)
    csum = lax.dot_general(ltri, mm, (((1,), (0,)), ((), ())),
                           preferred_element_type=_F32)
    rank = csum + carry[...]

    e0_ref[...] = a0
    e1_ref[...] = a1
    w0_ref[...] = jnp.broadcast_to(w0, (RT, WPAD))
    w1_ref[...] = jnp.broadcast_to(w1, (RT, WPAD))
    r0_ref[...] = jnp.sum(jnp.where(oh0, rank, 0.0), axis=1,
                          keepdims=True).astype(_I32)
    r1_ref[...] = jnp.sum(jnp.where(oh1, rank, 0.0), axis=1,
                          keepdims=True).astype(_I32)

    new_carry = carry[...] + jnp.sum(mm, axis=0, keepdims=True)
    carry[...] = new_carry

    @pl.when(i == NRT - 1)
    def _():
        cnt = new_carry.astype(_I32)
        padded = ((cnt + (BLK - 1)) >> BSH) << BSH
        pf = padded.astype(_F32)
        r8 = lax.broadcasted_iota(_I32, (NEXP, NEXP), 0)
        c8 = lax.broadcasted_iota(_I32, (NEXP, NEXP), 1)
        ut = (r8 < c8).astype(_F32)
        pstart = lax.dot_general(pf, ut, (((1,), (0,)), ((), ())),
                                 preferred_element_type=_F32)
        ps_ref[...] = pstart.astype(_I32)
        tb = (lax.broadcasted_iota(_I32, (NT, 1), 0) * BLK).astype(_F32)
        psb = jnp.broadcast_to(pstart, (NT, NEXP))
        acc = jnp.sum((psb <= tb).astype(_I32), axis=1, keepdims=True)
        map_ref[...] = jnp.clip(acc - 1, 0, NEXP - 1)


_router = pl.pallas_call(
    _router_body,
    grid=(NRT,),
    in_specs=[
        pl.BlockSpec((RT, HID), lambda i: (i, 0)),
        pl.BlockSpec((NEXP, HID), lambda i: (0, 0)),
    ],
    out_specs=[
        pl.BlockSpec((RT, 1), lambda i: (i, 0)),
        pl.BlockSpec((RT, 1), lambda i: (i, 0)),
        pl.BlockSpec((RT, WPAD), lambda i: (i, 0)),
        pl.BlockSpec((RT, WPAD), lambda i: (i, 0)),
        pl.BlockSpec((RT, 1), lambda i: (i, 0)),
        pl.BlockSpec((RT, 1), lambda i: (i, 0)),
        pl.BlockSpec((1, NEXP), lambda i: (0, 0)),
        pl.BlockSpec((NT, 1), lambda i: (0, 0)),
    ],
    out_shape=[
        jax.ShapeDtypeStruct((TOKENS, 1), _I32),
        jax.ShapeDtypeStruct((TOKENS, 1), _I32),
        jax.ShapeDtypeStruct((TOKENS, WPAD), _F32),
        jax.ShapeDtypeStruct((TOKENS, WPAD), _F32),
        jax.ShapeDtypeStruct((TOKENS, 1), _I32),
        jax.ShapeDtypeStruct((TOKENS, 1), _I32),
        jax.ShapeDtypeStruct((1, NEXP), _I32),
        jax.ShapeDtypeStruct((NT, 1), _I32),
    ],
    scratch_shapes=[pltpu.VMEM((1, NEXP), _F32)],
)


@functools.lru_cache(maxsize=None)
def _sc_kernels():
    mesh = plsc.VectorSubcoreMesh(core_axis_name="c", subcore_axis_name="s",
                                  num_cores=NC, num_subcores=NS)

    @functools.partial(
        pl.kernel,
        out_type=(
            jax.ShapeDtypeStruct((CAP, HID), _F32),
            jax.ShapeDtypeStruct((CAP, WPAD), _F32),
            jax.ShapeDtypeStruct((TOKENS,), _I32),
            jax.ShapeDtypeStruct((TOKENS,), _I32),
        ),
        mesh=mesh,
        scratch_types=[
            pltpu.VMEM((CHUNK,), _I32),
            pltpu.VMEM((CHUNK,), _I32),
            pltpu.VMEM((CHUNK,), _I32),
            pltpu.VMEM((CHUNK,), _I32),
            pltpu.VMEM((NEXP,), _I32),
            pltpu.VMEM((CHUNK, HID), _F32),
            pltpu.VMEM((CHUNK, WPAD), _F32),
            pltpu.VMEM((CHUNK, WPAD), _F32),
        ],
        compiler_params=pltpu.CompilerParams(needs_layout_passes=False),
    )
    def _dispatch(x_hbm, e0_hbm, e1_hbm, r0_hbm, r1_hbm, w0_hbm, w1_hbm,
                  ps_hbm, xs_hbm, ws_hbm, d0_hbm, d1_hbm,
                  e0_v, r0_v, idx0_v, idx1_v, ps_v, rows_v, w0_v, w1_v):
        wid = lax.axis_index("s") * NC + lax.axis_index("c")
        base = wid * CHUNK
        pltpu.sync_copy(ps_hbm, ps_v)
        pltpu.sync_copy(x_hbm.at[pl.ds(base, CHUNK), :], rows_v)
        pltpu.sync_copy(w0_hbm.at[pl.ds(base, CHUNK), :], w0_v)
        pltpu.sync_copy(w1_hbm.at[pl.ds(base, CHUNK), :], w1_v)

        pltpu.sync_copy(e0_hbm.at[pl.ds(base, CHUNK)], e0_v)
        pltpu.sync_copy(r0_hbm.at[pl.ds(base, CHUNK)], r0_v)
        for k in range(CHUNK // LANES):
            sl = pl.ds(k * LANES, LANES)
            idx0_v[sl] = plsc.load_gather(ps_v, [e0_v[sl]]) + r0_v[sl]
        pltpu.sync_copy(e1_hbm.at[pl.ds(base, CHUNK)], e0_v)
        pltpu.sync_copy(r1_hbm.at[pl.ds(base, CHUNK)], r0_v)
        for k in range(CHUNK // LANES):
            sl = pl.ds(k * LANES, LANES)
            idx1_v[sl] = plsc.load_gather(ps_v, [e0_v[sl]]) + r0_v[sl]

        pltpu.sync_copy(rows_v, xs_hbm.at[idx0_v])
        pltpu.sync_copy(rows_v, xs_hbm.at[idx1_v])
        pltpu.sync_copy(w0_v, ws_hbm.at[idx0_v])
        pltpu.sync_copy(w1_v, ws_hbm.at[idx1_v])
        pltpu.sync_copy(idx0_v, d0_hbm.at[pl.ds(base, CHUNK)])
        pltpu.sync_copy(idx1_v, d1_hbm.at[pl.ds(base, CHUNK)])

    @functools.partial(
        pl.kernel,
        out_type=jax.ShapeDtypeStruct((TOKENS, HID), _F32),
        mesh=mesh,
        scratch_types=[
            pltpu.VMEM((SUB,), _I32),
            pltpu.VMEM((SUB,), _I32),
            pltpu.VMEM((SUB, HID), _F32),
            pltpu.VMEM((SUB, HID), _F32),
        ],
    )
    def _combine(y_hbm, d0_hbm, d1_hbm, out_hbm,
                 idx0_v, idx1_v, rows0_v, rows1_v):
        wid = lax.axis_index("s") * NC + lax.axis_index("c")
        for h in range(CHUNK // SUB):
            base = wid * CHUNK + h * SUB
            pltpu.sync_copy(d0_hbm.at[pl.ds(base, SUB)], idx0_v)
            pltpu.sync_copy(d1_hbm.at[pl.ds(base, SUB)], idx1_v)
            pltpu.sync_copy(y_hbm.at[idx0_v], rows0_v)
            pltpu.sync_copy(y_hbm.at[idx1_v], rows1_v)

            def _tok(t, c):
                for j in range(HID // LANES):
                    sl = pl.ds(j * LANES, LANES)
                    rows0_v[t, sl] = rows0_v[t, sl] + rows1_v[t, sl]
                return c
            lax.fori_loop(0, SUB, _tok, 0)
            pltpu.sync_copy(rows0_v, out_hbm.at[pl.ds(base, SUB), :])

    return _dispatch, _combine


def _ffn_body(m_ref, xs_ref, wg_ref, wu_ref, wd_ref, ws_ref, y_ref):
    del m_ref
    x = xs_ref[...]
    g = jnp.dot(x, wg_ref[0], preferred_element_type=_F32)
    u = jnp.dot(x, wu_ref[0], preferred_element_type=_F32)
    inter = g * jax.nn.sigmoid(g) * u
    y = jnp.dot(inter, wd_ref[0], preferred_element_type=_F32)
    w = ws_ref[...][:, :1]
    y_ref[...] = y * w


_ffn = pl.pallas_call(
    _ffn_body,
    grid_spec=pltpu.PrefetchScalarGridSpec(
        num_scalar_prefetch=1,
        grid=(NT,),
        in_specs=[
            pl.BlockSpec((BLK, HID), lambda i, m: (i, 0)),
            pl.BlockSpec((1, HID, INTER), lambda i, m: (m[i], 0, 0)),
            pl.BlockSpec((1, HID, INTER), lambda i, m: (m[i], 0, 0)),
            pl.BlockSpec((1, INTER, HID), lambda i, m: (m[i], 0, 0)),
            pl.BlockSpec((BLK, WPAD), lambda i, m: (i, 0)),
        ],
        out_specs=pl.BlockSpec((BLK, HID), lambda i, m: (i, 0)),
    ),
    out_shape=jax.ShapeDtypeStruct((CAP, HID), _F32),
)


def kernel(hidden_states, gate_proj_w, gate_weights, up_weights, down_weights):
    dispatch, combine = _sc_kernels()
    e0, e1, w0b, w1b, r0, r1, ps, emap = _router(hidden_states, gate_proj_w)
    xs, ws, d0, d1 = dispatch(
        hidden_states, e0.reshape(TOKENS), e1.reshape(TOKENS),
        r0.reshape(TOKENS), r1.reshape(TOKENS), w0b, w1b, ps.reshape(NEXP))
    y = _ffn(emap.reshape(NT), xs, gate_weights, up_weights, down_weights, ws)
    return combine(y, d0, d1)
